# baseline (speedup 1.0000x reference)
import jax
import jax.numpy as jnp
from jax.experimental import pallas as pl

_K = 8


def _router_block_kernel(x_ref, w1_ref, b1_ref, w2_ref, b2_ref,
                         logits_ref, weights_ref, indices_ref, masks_ref):
    h = jnp.dot(x_ref[...], w1_ref[...], preferred_element_type=jnp.float32)
    h = h + b1_ref[...]
    logits = jnp.dot(h, w2_ref[...], preferred_element_type=jnp.float32)
    logits = logits + b2_ref[...]
    logits_ref[...] = logits

    nb, e = logits.shape
    iota_e = jax.lax.broadcasted_iota(jnp.int32, (nb, e), 1)
    work = logits
    vals, idxs = [], []
    for _ in range(_K):
        m = jnp.max(work, axis=-1, keepdims=True)
        idx = jnp.min(jnp.where(work == m, iota_e, e), axis=-1, keepdims=True)
        sel = iota_e == idx
        work = jnp.where(sel, -jnp.inf, work)
        vals.append(m)
        idxs.append(idx)
    vals = jnp.concatenate(vals, axis=1)
    idxs = jnp.concatenate(idxs, axis=1)

    w = jnp.exp(vals - vals[:, 0:1])
    weights_ref[...] = w / jnp.sum(w, axis=1, keepdims=True)
    indices_ref[...] = idxs

    idxs_t = jnp.transpose(idxs, (1, 0))
    iota_e2 = jax.lax.broadcasted_iota(jnp.int32, (e, nb), 0)
    for k in range(_K):
        mask_k = (iota_e2 == idxs_t[k:k + 1, :]).astype(jnp.int32)
        masks_ref[:, k, :] = mask_k


@jax.jit
def kernel(x, W1, b1, W2, b2):
    n, d = x.shape
    h_dim = W1.shape[1]
    e = W2.shape[1]
    nb = 512 if n % 512 == 0 else n
    grid = (n // nb,)
    out_shapes = (
        jax.ShapeDtypeStruct((n, e), jnp.float32),
        jax.ShapeDtypeStruct((n, _K), jnp.float32),
        jax.ShapeDtypeStruct((n, _K), jnp.int32),
        jax.ShapeDtypeStruct((e, _K, n), jnp.int32),
    )
    return pl.pallas_call(
        _router_block_kernel,
        grid=grid,
        in_specs=[
            pl.BlockSpec((nb, d), lambda i: (i, 0)),
            pl.BlockSpec((d, h_dim), lambda i: (0, 0)),
            pl.BlockSpec((1, h_dim), lambda i: (0, 0)),
            pl.BlockSpec((h_dim, e), lambda i: (0, 0)),
            pl.BlockSpec((1, e), lambda i: (0, 0)),
        ],
        out_specs=(
            pl.BlockSpec((nb, e), lambda i: (i, 0)),
            pl.BlockSpec((nb, _K), lambda i: (i, 0)),
            pl.BlockSpec((nb, _K), lambda i: (i, 0)),
            pl.BlockSpec((e, _K, nb), lambda i: (0, 0, i)),
        ),
        out_shape=out_shapes,
    )(x, W1, b1.reshape(1, -1), W2, b2.reshape(1, -1))


# speedup vs baseline: 2.4590x; 2.4590x over previous
import jax
import jax.numpy as jnp
from jax.experimental import pallas as pl

_K = 8


def _router_block_kernel(x_ref, w1_ref, b1_ref, w2_ref, b2_ref,
                         logits_ref, weights_ref, indices_ref, masks_ref):
    h = jnp.dot(x_ref[...], w1_ref[...], preferred_element_type=jnp.float32)
    h = h + b1_ref[...]
    logits = jnp.dot(h, w2_ref[...], preferred_element_type=jnp.float32)
    logits = logits + b2_ref[...]
    logits_ref[...] = logits

    lt = logits.T
    e, nb = lt.shape
    iota_s = jax.lax.broadcasted_iota(jnp.int32, (e, nb), 0)
    work = lt
    vals, idxs = [], []
    for _ in range(_K):
        m = jnp.max(work, axis=0, keepdims=True)
        hit = work == m
        idx = jnp.min(jnp.where(hit, iota_s, e), axis=0, keepdims=True)
        sel = iota_s == idx
        work = jnp.where(sel, -jnp.inf, work)
        vals.append(m)
        idxs.append(idx)
    vals_t = jnp.concatenate(vals, axis=0)
    idxs_t = jnp.concatenate(idxs, axis=0)

    w = jnp.exp(vals_t - vals_t[0:1])
    wn = w / jnp.sum(w, axis=0, keepdims=True)
    weights_ref[...] = wn.T
    indices_ref[...] = idxs_t.T

    iota_e3 = jax.lax.broadcasted_iota(jnp.int32, (e, _K, nb), 0)
    masks_ref[...] = (iota_e3 == idxs_t[None, :, :]).astype(jnp.int32)


@jax.jit
def kernel(x, W1, b1, W2, b2):
    n, d = x.shape
    h_dim = W1.shape[1]
    e = W2.shape[1]
    nb = 1024 if n % 1024 == 0 else n
    grid = (n // nb,)
    out_shapes = (
        jax.ShapeDtypeStruct((n, e), jnp.float32),
        jax.ShapeDtypeStruct((n, _K), jnp.float32),
        jax.ShapeDtypeStruct((n, _K), jnp.int32),
        jax.ShapeDtypeStruct((e, _K, n), jnp.int32),
    )
    return pl.pallas_call(
        _router_block_kernel,
        grid=grid,
        in_specs=[
            pl.BlockSpec((nb, d), lambda i: (i, 0)),
            pl.BlockSpec((d, h_dim), lambda i: (0, 0)),
            pl.BlockSpec((1, h_dim), lambda i: (0, 0)),
            pl.BlockSpec((h_dim, e), lambda i: (0, 0)),
            pl.BlockSpec((1, e), lambda i: (0, 0)),
        ],
        out_specs=(
            pl.BlockSpec((nb, e), lambda i: (i, 0)),
            pl.BlockSpec((nb, _K), lambda i: (i, 0)),
            pl.BlockSpec((nb, _K), lambda i: (i, 0)),
            pl.BlockSpec((e, _K, nb), lambda i: (0, 0, i)),
        ),
        out_shape=out_shapes,
    )(x, W1, b1.reshape(1, -1), W2, b2.reshape(1, -1))
